# Initial kernel scaffold; baseline (speedup 1.0000x reference)
import functools

import jax
import jax.numpy as jnp
from jax import lax
from jax.experimental import pallas as pl
from jax.experimental.pallas import tpu as pltpu
from jax.experimental.pallas import tpu_sc as plsc

N = 10000
DF = 128
NCLS = 64
NPAD = 10240
DE = 144
CH = 128
NC, NS = 2, 16
NW = NC * NS
ROWS_PER_TILE = NPAD // NS


def _sc_aggregate(x_ext, src_idx, dst_idx, zeros, cpt):
    mesh = plsc.VectorSubcoreMesh(core_axis_name="c", subcore_axis_name="s")

    @functools.partial(
        pl.kernel,
        out_type=jax.ShapeDtypeStruct((NC, NPAD, DE), jnp.float32),
        mesh=mesh,
        scratch_types=[
            pltpu.VMEM_SHARED((NPAD, DE), jnp.float32),
            pltpu.VMEM((cpt, CH), jnp.int32),
            pltpu.VMEM((cpt, CH), jnp.int32),
            pltpu.VMEM((CH, DE), jnp.float32),
        ],
    )
    def k(x_hbm, src_hbm, dst_hbm, z_hbm, out_hbm, acc, src_v, dst_v, rows_v):
        c = lax.axis_index("c")
        s = lax.axis_index("s")
        wid = c * NS + s
        pltpu.sync_copy(z_hbm, acc.at[pl.ds(s * ROWS_PER_TILE, ROWS_PER_TILE)])
        pltpu.sync_copy(src_hbm.at[wid], src_v)
        pltpu.sync_copy(dst_hbm.at[wid], dst_v)
        plsc.subcore_barrier()

        @pl.loop(0, cpt)
        def _(j):
            pltpu.sync_copy(x_hbm.at[src_v.at[j]], rows_v)
            pltpu.sync_copy(rows_v, acc.at[dst_v.at[j]], add=True)

        plsc.subcore_barrier()
        pltpu.sync_copy(acc.at[pl.ds(s * ROWS_PER_TILE, ROWS_PER_TILE)],
                        out_hbm.at[c, pl.ds(s * ROWS_PER_TILE, ROWS_PER_TILE)])

    return k(x_ext, src_idx, dst_idx, zeros)


BLK = 1000


def _tc_head(parts, x, wl_t, b_l, wr_t, wlin_t, b_lin):
    def body(p_ref, x_ref, wl_ref, bl_ref, wr_ref, wlin_ref, blin_ref,
             emb_ref, log_ref):
        p = p_ref[...]
        ssum = p[0] + p[1]
        agg = ssum[:, :DF]
        cnt = ssum[:, DF:DF + 1]
        mean = agg / jnp.maximum(cnt, 1.0)
        emb = (jnp.dot(mean, wl_ref[...], preferred_element_type=jnp.float32)
               + bl_ref[...]
               + jnp.dot(x_ref[...], wr_ref[...],
                         preferred_element_type=jnp.float32))
        emb_ref[...] = emb
        h = jnp.maximum(emb, 0.0)
        log_ref[...] = (jnp.dot(h, wlin_ref[...],
                                preferred_element_type=jnp.float32)
                        + blin_ref[...])

    return pl.pallas_call(
        body,
        grid=(N // BLK,),
        in_specs=[
            pl.BlockSpec((2, BLK, DE), lambda i: (0, i, 0)),
            pl.BlockSpec((BLK, DF), lambda i: (i, 0)),
            pl.BlockSpec((DF, DF), lambda i: (0, 0)),
            pl.BlockSpec((1, DF), lambda i: (0, 0)),
            pl.BlockSpec((DF, DF), lambda i: (0, 0)),
            pl.BlockSpec((DF, NCLS), lambda i: (0, 0)),
            pl.BlockSpec((1, NCLS), lambda i: (0, 0)),
        ],
        out_specs=[
            pl.BlockSpec((BLK, DF), lambda i: (i, 0)),
            pl.BlockSpec((BLK, NCLS), lambda i: (i, 0)),
        ],
        out_shape=[
            jax.ShapeDtypeStruct((N, DF), jnp.float32),
            jax.ShapeDtypeStruct((N, NCLS), jnp.float32),
        ],
    )(parts, x, wl_t, b_l, wr_t, wlin_t, b_lin)


def kernel(x, edge_index, W_l, b_l, W_r, W_lin, b_lin):
    e = edge_index.shape[1]
    cpt = -(-e // (CH * NW))
    e_pad = cpt * CH * NW
    src = edge_index[0].astype(jnp.int32)
    dst = edge_index[1].astype(jnp.int32)
    fill = jnp.full((e_pad - e,), N, dtype=jnp.int32)
    src_p = jnp.concatenate([src, fill]).reshape(NW, cpt, CH)
    dst_p = jnp.concatenate([dst, fill]).reshape(NW, cpt, CH)
    x_ext = jnp.zeros((NPAD, DE), jnp.float32)
    x_ext = x_ext.at[:N, :DF].set(x).at[:N, DF].set(1.0)
    zeros = jnp.zeros((ROWS_PER_TILE, DE), jnp.float32)

    parts = _sc_aggregate(x_ext, src_p, dst_p, zeros, cpt)
    emb, logits = _tc_head(parts, x, W_l.T, b_l.reshape(1, DF), W_r.T,
                           W_lin.T, b_lin.reshape(1, NCLS))
    return (emb, logits)


# speedup vs baseline: 4.6739x; 4.6739x over previous
import functools

import jax
import jax.numpy as jnp
from jax import lax
from jax.experimental import pallas as pl
from jax.experimental.pallas import tpu as pltpu
from jax.experimental.pallas import tpu_sc as plsc

N = 10000
DF = 128
NCLS = 64
NPAD = 10240
DE = 144
CH = 128
NC, NS = 2, 16
NW = NC * NS
ROWS_PER_TILE = NPAD // NS


def _sc_aggregate(x_ext, src_idx, dst_idx, zeros, cpt):
    mesh = plsc.VectorSubcoreMesh(core_axis_name="c", subcore_axis_name="s")

    @functools.partial(
        pl.kernel,
        out_type=jax.ShapeDtypeStruct((NC, NPAD, DE), jnp.float32),
        mesh=mesh,
        scratch_types=[
            pltpu.VMEM_SHARED((NPAD, DE), jnp.float32),
            pltpu.VMEM((cpt, CH), jnp.int32),
            pltpu.VMEM((cpt, CH), jnp.int32),
            pltpu.VMEM((CH, DE), jnp.float32),
        ],
        compiler_params=pltpu.CompilerParams(use_tc_tiling_on_sc=False),
    )
    def k(x_hbm, src_hbm, dst_hbm, z_hbm, out_hbm, acc, src_v, dst_v, rows_v):
        c = lax.axis_index("c")
        s = lax.axis_index("s")
        wid = c * NS + s
        pltpu.sync_copy(z_hbm, acc.at[pl.ds(s * ROWS_PER_TILE, ROWS_PER_TILE)])
        pltpu.sync_copy(src_hbm.at[wid], src_v)
        pltpu.sync_copy(dst_hbm.at[wid], dst_v)
        plsc.subcore_barrier()

        @pl.loop(0, cpt)
        def _(j):
            pltpu.sync_copy(x_hbm.at[src_v.at[j]], rows_v)
            pltpu.sync_copy(rows_v, acc.at[dst_v.at[j]], add=True)

        plsc.subcore_barrier()
        pltpu.sync_copy(acc.at[pl.ds(s * ROWS_PER_TILE, ROWS_PER_TILE)],
                        out_hbm.at[c, pl.ds(s * ROWS_PER_TILE, ROWS_PER_TILE)])

    return k(x_ext, src_idx, dst_idx, zeros)


BLK = 1000


def _tc_head(parts, x, wl_t, b_l, wr_t, wlin_t, b_lin):
    def body(p_ref, x_ref, wl_ref, bl_ref, wr_ref, wlin_ref, blin_ref,
             emb_ref, log_ref):
        p = p_ref[...]
        ssum = p[0] + p[1]
        agg = ssum[:, :DF]
        cnt = ssum[:, DF:DF + 1]
        mean = agg / jnp.maximum(cnt, 1.0)
        emb = (jnp.dot(mean, wl_ref[...], preferred_element_type=jnp.float32)
               + bl_ref[...]
               + jnp.dot(x_ref[...], wr_ref[...],
                         preferred_element_type=jnp.float32))
        emb_ref[...] = emb
        h = jnp.maximum(emb, 0.0)
        log_ref[...] = (jnp.dot(h, wlin_ref[...],
                                preferred_element_type=jnp.float32)
                        + blin_ref[...])

    return pl.pallas_call(
        body,
        grid=(N // BLK,),
        in_specs=[
            pl.BlockSpec((2, BLK, DE), lambda i: (0, i, 0)),
            pl.BlockSpec((BLK, DF), lambda i: (i, 0)),
            pl.BlockSpec((DF, DF), lambda i: (0, 0)),
            pl.BlockSpec((1, DF), lambda i: (0, 0)),
            pl.BlockSpec((DF, DF), lambda i: (0, 0)),
            pl.BlockSpec((DF, NCLS), lambda i: (0, 0)),
            pl.BlockSpec((1, NCLS), lambda i: (0, 0)),
        ],
        out_specs=[
            pl.BlockSpec((BLK, DF), lambda i: (i, 0)),
            pl.BlockSpec((BLK, NCLS), lambda i: (i, 0)),
        ],
        out_shape=[
            jax.ShapeDtypeStruct((N, DF), jnp.float32),
            jax.ShapeDtypeStruct((N, NCLS), jnp.float32),
        ],
    )(parts, x, wl_t, b_l, wr_t, wlin_t, b_lin)


def kernel(x, edge_index, W_l, b_l, W_r, W_lin, b_lin):
    e = edge_index.shape[1]
    cpt = -(-e // (CH * NW))
    e_pad = cpt * CH * NW
    src = edge_index[0].astype(jnp.int32)
    dst = edge_index[1].astype(jnp.int32)
    fill = jnp.full((e_pad - e,), N, dtype=jnp.int32)
    src_p = jnp.concatenate([src, fill]).reshape(NW, cpt, CH)
    dst_p = jnp.concatenate([dst, fill]).reshape(NW, cpt, CH)
    x_ext = jnp.zeros((NPAD, DE), jnp.float32)
    x_ext = x_ext.at[:N, :DF].set(x).at[:N, DF].set(1.0)
    zeros = jnp.zeros((ROWS_PER_TILE, DE), jnp.float32)

    parts = _sc_aggregate(x_ext, src_p, dst_p, zeros, cpt)
    emb, logits = _tc_head(parts, x, W_l.T, b_l.reshape(1, DF), W_r.T,
                           W_lin.T, b_lin.reshape(1, NCLS))
    return (emb, logits)
